# baseline (speedup 1.0000x reference)
import functools

import jax
import jax.numpy as jnp
from jax import lax
from jax.experimental import pallas as pl
from jax.experimental.pallas import tpu as pltpu
from jax.experimental.pallas import tpu_sc as plsc

N = 10000
E = 320000
D_IN = 128
D_H = 16

NC = 2
NS = 16
NW = NC * NS
CHUNK = 128
CPT = 80
E_PAD = NW * CPT * CHUNK
NROWS = E_PAD // CHUNK
N_PAD = 10240
SLICE = N_PAD // NS
PAD_ROWS = 240
GRP = 8
NGRP = CPT // GRP
NP8 = N_PAD // 8
PSL = NP8 // NS
ROWS1 = 128
GRID1 = NP8 // ROWS1
ROWS_TC = 256
GRID_TC = NP8 // ROWS_TC
NOUT8 = N // 8

_mesh = plsc.VectorSubcoreMesh(core_axis_name="c", subcore_axis_name="s")


@functools.partial(
    pl.kernel,
    out_type=jax.ShapeDtypeStruct((NC * NP8 * 128,), jnp.float32),
    mesh=_mesh,
    scratch_types=[
        pltpu.VMEM((CPT, CHUNK), jnp.int32),
        pltpu.VMEM((CHUNK,), jnp.float32),
        pltpu.VMEM((SLICE,), jnp.float32),
        pltpu.VMEM((SLICE * 16,), jnp.float32),
        pltpu.VMEM_SHARED((N_PAD,), jnp.float32),
        pltpu.SemaphoreType.DMA,
    ],
)
def _sc_degree(dstm, out_hbm, dst_v, ones_v, stage, packed, acc, ssem):
    c = lax.axis_index("c")
    s = lax.axis_index("s")
    tid = c * NS + s
    pltpu.sync_copy(dstm.at[pl.ds(tid * CPT, CPT)], dst_v)

    def _fill(i, _):
        ones_v[pl.ds(i * 16, 16)] = jnp.ones((16,), jnp.float32)
        return 0

    lax.fori_loop(0, CHUNK // 16, _fill, 0)

    def _zero(i, _):
        stage[pl.ds(i * 16, 16)] = jnp.zeros((16,), jnp.float32)
        return 0

    lax.fori_loop(0, SLICE // 16, _zero, 0)
    pltpu.sync_copy(stage, acc.at[pl.ds(s * SLICE, SLICE)])
    plsc.subcore_barrier()

    def _group(gi, _):
        for b in range(8):
            pltpu.async_copy(ones_v, acc.at[dst_v.at[gi * 8 + b]], ssem,
                             add=True)
        for b in range(8):
            pltpu.make_async_copy(ones_v, acc.at[dst_v.at[gi * 8 + b]],
                                  ssem).wait()
        return 0

    lax.fori_loop(0, CPT // 8, _group, 0)
    plsc.subcore_barrier()
    pltpu.sync_copy(acc.at[pl.ds(s * SLICE, SLICE)], stage)

    def _expand(k, _):
        v = stage[pl.ds(k * 16, 16)]
        for l in range(16):
            bcast = v.at[jnp.full((16,), l, jnp.int32)].get(
                mode="promise_in_bounds")
            packed[pl.ds((k * 16 + l) * 16, 16)] = bcast
        return 0

    lax.fori_loop(0, SLICE // 16, _expand, 0)
    pltpu.sync_copy(packed,
                    out_hbm.at[pl.ds((c * N_PAD + s * SLICE) * 16, SLICE * 16)])


@functools.partial(
    pl.kernel,
    out_type=jax.ShapeDtypeStruct((NC * N_PAD, D_H), jnp.float32),
    mesh=_mesh,
    compiler_params=pltpu.CompilerParams(use_tc_tiling_on_sc=False),
    scratch_types=[
        pltpu.VMEM((CPT, CHUNK), jnp.int32),
        pltpu.VMEM((CPT, CHUNK), jnp.int32),
        [pltpu.VMEM((CHUNK, D_H), jnp.float32) for _ in range(2 * GRP)],
        pltpu.VMEM((SLICE, D_H), jnp.float32),
        pltpu.VMEM_SHARED((N_PAD, D_H), jnp.float32),
        [pltpu.SemaphoreType.DMA for _ in range(4)],
    ],
)
def _sc_propagate(g_hbm, srcm, dstm, out_hbm, src_v, dst_v, bufs, stage, acc, sems):
    c = lax.axis_index("c")
    s = lax.axis_index("s")
    tid = c * NS + s
    row0 = tid * CPT
    pltpu.sync_copy(srcm.at[pl.ds(row0, CPT)], src_v)
    pltpu.sync_copy(dstm.at[pl.ds(row0, CPT)], dst_v)

    def _zero(i, _):
        stage[i, :] = jnp.zeros((16,), jnp.float32)
        return 0

    lax.fori_loop(0, SLICE, _zero, 0)
    pltpu.sync_copy(stage, acc.at[pl.ds(s * SLICE, SLICE)])
    plsc.subcore_barrier()

    gsem = [sems[0], sems[1]]
    ssem = [sems[2], sems[3]]

    def _gather(j, buf, sem):
        pltpu.async_copy(g_hbm.at[src_v.at[j]], buf, sem)

    def _gwait(j, buf, sem):
        pltpu.make_async_copy(g_hbm.at[src_v.at[j]], buf, sem).wait()

    for b in range(GRP):
        _gather(b, bufs[b], gsem[0])
        _gather(GRP + b, bufs[GRP + b], gsem[1])

    def _group(m, _):
        for p in range(2):
            g0 = 2 * m + p
            sbufs = bufs[p * GRP:(p + 1) * GRP]
            for b in range(GRP):
                _gwait(g0 * GRP + b, sbufs[b], gsem[p])
            for b in range(GRP):
                pltpu.async_copy(sbufs[b], acc.at[dst_v.at[g0 * GRP + b]],
                                 ssem[p], add=True)
            for b in range(GRP):
                pltpu.make_async_copy(sbufs[b],
                                      acc.at[dst_v.at[g0 * GRP + b]],
                                      ssem[p]).wait()

            @pl.when(g0 + 2 < NGRP)
            def _():
                for b in range(GRP):
                    _gather((g0 + 2) * GRP + b, sbufs[b], gsem[p])

        return 0

    lax.fori_loop(0, NGRP // 2, _group, 0)
    plsc.subcore_barrier()
    pltpu.sync_copy(acc.at[pl.ds(s * SLICE, SLICE)], stage)
    pltpu.sync_copy(stage, out_hbm.at[pl.ds(c * N_PAD + s * SLICE, SLICE)])


def _tc_mm1(x_p, w1w, mask_big, ssum, b1b):
    def body(x_ref, w_ref, m_ref, s_ref, b_ref, h_ref):
        hw = jnp.dot(x_ref[...], w_ref[...], preferred_element_type=jnp.float32)
        hp = jnp.dot(s_ref[...], hw * m_ref[...],
                     preferred_element_type=jnp.float32)
        h_ref[...] = hp + b_ref[...]

    return pl.pallas_call(
        body,
        grid=(GRID1,),
        in_specs=[
            pl.BlockSpec((8 * ROWS1, D_IN), lambda i: (i, 0)),
            pl.BlockSpec((D_IN, 128), lambda i: (0, 0)),
            pl.BlockSpec((8 * ROWS1, 128), lambda i: (0, 0)),
            pl.BlockSpec((ROWS1, 8 * ROWS1), lambda i: (0, 0)),
            pl.BlockSpec((1, 128), lambda i: (0, 0)),
        ],
        out_specs=pl.BlockSpec((ROWS1, 128), lambda i: (i, 0)),
        out_shape=jax.ShapeDtypeStruct((NP8, 128), jnp.float32),
    )(x_p, w1w, mask_big, ssum, b1b)


def _tc_scale1(degp, h1p):
    def body(d_ref, h_ref, g_ref, dinv_ref):
        deg = d_ref[0] + d_ref[1]
        dinv = jnp.where(deg > 0.0, lax.rsqrt(deg), 0.0)
        g_ref[...] = dinv * h_ref[...]
        dinv_ref[...] = dinv

    return pl.pallas_call(
        body,
        grid=(GRID_TC,),
        in_specs=[
            pl.BlockSpec((NC, ROWS_TC, 128), lambda i: (0, i, 0)),
            pl.BlockSpec((ROWS_TC, 128), lambda i: (i, 0)),
        ],
        out_specs=[
            pl.BlockSpec((ROWS_TC, 128), lambda i: (i, 0)),
            pl.BlockSpec((ROWS_TC, 128), lambda i: (i, 0)),
        ],
        out_shape=[
            jax.ShapeDtypeStruct((NP8, 128), jnp.float32),
            jax.ShapeDtypeStruct((NP8, 128), jnp.float32),
        ],
    )(degp, h1p)


def _tc_layer2(sp, dinvp, w2b, b2b):
    def body(s_ref, dinv_ref, w_ref, b_ref, g_ref):
        a = jax.nn.sigmoid(dinv_ref[...] * (s_ref[0] + s_ref[1]))
        h = jnp.dot(a, w_ref[...], preferred_element_type=jnp.float32)
        g_ref[...] = dinv_ref[...] * (h + b_ref[...])

    return pl.pallas_call(
        body,
        grid=(GRID_TC,),
        in_specs=[
            pl.BlockSpec((NC, ROWS_TC, 128), lambda i: (0, i, 0)),
            pl.BlockSpec((ROWS_TC, 128), lambda i: (i, 0)),
            pl.BlockSpec((128, 128), lambda i: (0, 0)),
            pl.BlockSpec((1, 128), lambda i: (0, 0)),
        ],
        out_specs=pl.BlockSpec((ROWS_TC, 128), lambda i: (i, 0)),
        out_shape=jax.ShapeDtypeStruct((NP8, 128), jnp.float32),
    )(sp, dinvp, w2b, b2b)


def _tc_out(sp, dinvp):
    def body(s_ref, dinv_ref, o_ref):
        o_ref[...] = jax.nn.sigmoid(dinv_ref[...] * (s_ref[0] + s_ref[1]))

    return pl.pallas_call(
        body,
        grid=(GRID_TC,),
        in_specs=[
            pl.BlockSpec((NC, ROWS_TC, 128), lambda i: (0, i, 0)),
            pl.BlockSpec((ROWS_TC, 128), lambda i: (i, 0)),
        ],
        out_specs=pl.BlockSpec((ROWS_TC, 128), lambda i: (i, 0)),
        out_shape=jax.ShapeDtypeStruct((NP8, 128), jnp.float32),
    )(sp, dinvp)


def kernel(x, edge_index, W1, b1, W2, b2):
    padm = (N + (jnp.arange(E_PAD - E, dtype=jnp.int32) % PAD_ROWS)).reshape(
        NROWS - E // CHUNK, CHUNK)
    dstm = jnp.concatenate([edge_index[1].reshape(E // CHUNK, CHUNK), padm])
    ei_b = lax.optimization_barrier(edge_index)
    srcm = jnp.concatenate([ei_b[0].reshape(E // CHUNK, CHUNK), padm])
    x_p = jnp.pad(x, ((0, N_PAD - N), (0, 0)))
    w1w = jnp.tile(W1.T, (1, 8))
    col = lax.broadcasted_iota(jnp.int32, (8 * ROWS1, 128), 1)
    row = lax.broadcasted_iota(jnp.int32, (8 * ROWS1, 128), 0)
    mask_big = (col // 16 == row % 8).astype(jnp.float32)
    sr = lax.broadcasted_iota(jnp.int32, (ROWS1, 8 * ROWS1), 0)
    sc = lax.broadcasted_iota(jnp.int32, (ROWS1, 8 * ROWS1), 1)
    ssum = (sc // 8 == sr).astype(jnp.float32)
    bc = lax.broadcasted_iota(jnp.int32, (128, 128), 1)
    br = lax.broadcasted_iota(jnp.int32, (128, 128), 0)
    w2b = jnp.tile(W2.T, (8, 8)) * (bc // 16 == br // 16).astype(jnp.float32)
    b1b = jnp.tile(b1, 8)[None, :]
    b2b = jnp.tile(b2, 8)[None, :]

    degp = _sc_degree(dstm).reshape(NC, NP8, 128)
    h1p = _tc_mm1(x_p, w1w, mask_big, ssum, b1b)
    g1p, dinvp = _tc_scale1(degp, h1p)
    s1p = _sc_propagate(g1p.reshape(N_PAD, D_H), srcm, dstm).reshape(
        NC, NP8, 128)
    g2p = _tc_layer2(s1p, dinvp, w2b, b2b)
    s2p = _sc_propagate(g2p.reshape(N_PAD, D_H), srcm, dstm).reshape(
        NC, NP8, 128)
    outp = _tc_out(s2p, dinvp)
    return outp[:NOUT8].reshape(N, D_H)


# speedup vs baseline: 1.1870x; 1.1870x over previous
import functools

import jax
import jax.numpy as jnp
from jax import lax
from jax.experimental import pallas as pl
from jax.experimental.pallas import tpu as pltpu
from jax.experimental.pallas import tpu_sc as plsc

N = 10000
E = 320000
D_IN = 128
D_H = 16

NC = 2
NS = 16
NW = NC * NS
CHUNK = 128
CPT = 80
E_PAD = NW * CPT * CHUNK
NROWS = E_PAD // CHUNK
N_PAD = 10240
SLICE = N_PAD // NS
PAD_ROWS = 240
GRP = 8
NGRP = CPT // GRP
NP8 = N_PAD // 8
PSL = NP8 // NS
ROWS1 = 128
GRID1 = NP8 // ROWS1
ROWS_TC = 256
GRID_TC = NP8 // ROWS_TC
NOUT8 = N // 8

_mesh = plsc.VectorSubcoreMesh(core_axis_name="c", subcore_axis_name="s")


@functools.partial(
    pl.kernel,
    out_type=jax.ShapeDtypeStruct((NC * NP8 * 128,), jnp.float32),
    mesh=_mesh,
    scratch_types=[
        pltpu.VMEM((CPT, CHUNK), jnp.int32),
        pltpu.VMEM((CHUNK,), jnp.float32),
        pltpu.VMEM((SLICE,), jnp.float32),
        pltpu.VMEM((SLICE * 16,), jnp.float32),
        pltpu.VMEM_SHARED((N_PAD,), jnp.float32),
        pltpu.SemaphoreType.DMA,
    ],
)
def _sc_degree(dstm, out_hbm, dst_v, ones_v, stage, packed, acc, ssem):
    c = lax.axis_index("c")
    s = lax.axis_index("s")
    tid = c * NS + s
    pltpu.sync_copy(dstm.at[pl.ds(tid * CPT, CPT)], dst_v)

    def _fill(i, _):
        ones_v[pl.ds(i * 16, 16)] = jnp.ones((16,), jnp.float32)
        return 0

    lax.fori_loop(0, CHUNK // 16, _fill, 0)

    def _zero(i, _):
        stage[pl.ds(i * 16, 16)] = jnp.zeros((16,), jnp.float32)
        return 0

    lax.fori_loop(0, SLICE // 16, _zero, 0)
    pltpu.sync_copy(stage, acc.at[pl.ds(s * SLICE, SLICE)])
    plsc.subcore_barrier()

    def _group(gi, _):
        for b in range(8):
            pltpu.async_copy(ones_v, acc.at[dst_v.at[gi * 8 + b]], ssem,
                             add=True)
        for b in range(8):
            pltpu.make_async_copy(ones_v, acc.at[dst_v.at[gi * 8 + b]],
                                  ssem).wait()
        return 0

    lax.fori_loop(0, CPT // 8, _group, 0)
    plsc.subcore_barrier()
    pltpu.sync_copy(acc.at[pl.ds(s * SLICE, SLICE)], stage)

    def _expand(k, _):
        v = stage[pl.ds(k * 16, 16)]
        for l in range(16):
            bcast = v.at[jnp.full((16,), l, jnp.int32)].get(
                mode="promise_in_bounds")
            packed[pl.ds((k * 16 + l) * 16, 16)] = bcast
        return 0

    lax.fori_loop(0, SLICE // 16, _expand, 0)
    pltpu.sync_copy(packed,
                    out_hbm.at[pl.ds((c * N_PAD + s * SLICE) * 16, SLICE * 16)])


@functools.partial(
    pl.kernel,
    out_type=jax.ShapeDtypeStruct((NC * N_PAD, D_H), jnp.float32),
    mesh=_mesh,
    compiler_params=pltpu.CompilerParams(use_tc_tiling_on_sc=False),
    scratch_types=[
        pltpu.VMEM((CPT, CHUNK), jnp.int32),
        pltpu.VMEM((CPT, CHUNK), jnp.int32),
        [pltpu.VMEM((CHUNK, D_H), jnp.float32) for _ in range(2 * GRP)],
        pltpu.VMEM((SLICE, D_H), jnp.float32),
        pltpu.VMEM_SHARED((N_PAD, D_H), jnp.float32),
        pltpu.VMEM_SHARED((N_PAD, D_H), jnp.float32),
        [pltpu.SemaphoreType.DMA for _ in range(4)],
    ],
)
def _sc_propagate(g_hbm, srcm, dstm, out_hbm, src_v, dst_v, bufs, stage, acc,
                  g_sh, sems):
    c = lax.axis_index("c")
    s = lax.axis_index("s")
    tid = c * NS + s
    row0 = tid * CPT
    pltpu.sync_copy(srcm.at[pl.ds(row0, CPT)], src_v)
    pltpu.sync_copy(dstm.at[pl.ds(row0, CPT)], dst_v)
    pltpu.sync_copy(g_hbm.at[pl.ds(s * SLICE, SLICE)],
                    g_sh.at[pl.ds(s * SLICE, SLICE)])

    def _zero(i, _):
        stage[i, :] = jnp.zeros((16,), jnp.float32)
        return 0

    lax.fori_loop(0, SLICE, _zero, 0)
    pltpu.sync_copy(stage, acc.at[pl.ds(s * SLICE, SLICE)])
    plsc.subcore_barrier()

    gsem = [sems[0], sems[1]]
    ssem = [sems[2], sems[3]]

    def _gather(j, buf, sem):
        pltpu.async_copy(g_sh.at[src_v.at[j]], buf, sem)

    def _gwait(j, buf, sem):
        pltpu.make_async_copy(g_sh.at[src_v.at[j]], buf, sem).wait()

    for b in range(GRP):
        _gather(b, bufs[b], gsem[0])
        _gather(GRP + b, bufs[GRP + b], gsem[1])

    def _group(m, _):
        for p in range(2):
            g0 = 2 * m + p
            sbufs = bufs[p * GRP:(p + 1) * GRP]
            for b in range(GRP):
                _gwait(g0 * GRP + b, sbufs[b], gsem[p])
            for b in range(GRP):
                pltpu.async_copy(sbufs[b], acc.at[dst_v.at[g0 * GRP + b]],
                                 ssem[p], add=True)
            for b in range(GRP):
                pltpu.make_async_copy(sbufs[b],
                                      acc.at[dst_v.at[g0 * GRP + b]],
                                      ssem[p]).wait()

            @pl.when(g0 + 2 < NGRP)
            def _():
                for b in range(GRP):
                    _gather((g0 + 2) * GRP + b, sbufs[b], gsem[p])

        return 0

    lax.fori_loop(0, NGRP // 2, _group, 0)
    plsc.subcore_barrier()
    pltpu.sync_copy(acc.at[pl.ds(s * SLICE, SLICE)], stage)
    pltpu.sync_copy(stage, out_hbm.at[pl.ds(c * N_PAD + s * SLICE, SLICE)])


def _tc_layer1(x_p, degp, w1w, mask_big, ssum, b1b):
    def body(x_ref, d_ref, w_ref, m_ref, s_ref, b_ref, g_ref, dinv_ref):
        deg = d_ref[0] + d_ref[1]
        dinv = jnp.where(deg > 0.0, lax.rsqrt(deg), 0.0)
        hw = jnp.dot(x_ref[...], w_ref[...], preferred_element_type=jnp.float32)
        hp = jnp.dot(s_ref[...], hw * m_ref[...],
                     preferred_element_type=jnp.float32)
        g_ref[...] = dinv * (hp + b_ref[...])
        dinv_ref[...] = dinv

    return pl.pallas_call(
        body,
        grid=(GRID1,),
        in_specs=[
            pl.BlockSpec((8 * ROWS1, D_IN), lambda i: (i, 0)),
            pl.BlockSpec((NC, ROWS1, 128), lambda i: (0, i, 0)),
            pl.BlockSpec((D_IN, 128), lambda i: (0, 0)),
            pl.BlockSpec((8 * ROWS1, 128), lambda i: (0, 0)),
            pl.BlockSpec((ROWS1, 8 * ROWS1), lambda i: (0, 0)),
            pl.BlockSpec((1, 128), lambda i: (0, 0)),
        ],
        out_specs=[
            pl.BlockSpec((ROWS1, 128), lambda i: (i, 0)),
            pl.BlockSpec((ROWS1, 128), lambda i: (i, 0)),
        ],
        out_shape=[
            jax.ShapeDtypeStruct((NP8, 128), jnp.float32),
            jax.ShapeDtypeStruct((NP8, 128), jnp.float32),
        ],
    )(x_p, degp, w1w, mask_big, ssum, b1b)


def _tc_layer2(sp, dinvp, w2b, b2b):
    def body(s_ref, dinv_ref, w_ref, b_ref, g_ref):
        a = jax.nn.sigmoid(dinv_ref[...] * (s_ref[0] + s_ref[1]))
        h = jnp.dot(a, w_ref[...], preferred_element_type=jnp.float32)
        g_ref[...] = dinv_ref[...] * (h + b_ref[...])

    return pl.pallas_call(
        body,
        grid=(GRID_TC,),
        in_specs=[
            pl.BlockSpec((NC, ROWS_TC, 128), lambda i: (0, i, 0)),
            pl.BlockSpec((ROWS_TC, 128), lambda i: (i, 0)),
            pl.BlockSpec((128, 128), lambda i: (0, 0)),
            pl.BlockSpec((1, 128), lambda i: (0, 0)),
        ],
        out_specs=pl.BlockSpec((ROWS_TC, 128), lambda i: (i, 0)),
        out_shape=jax.ShapeDtypeStruct((NP8, 128), jnp.float32),
    )(sp, dinvp, w2b, b2b)


def _tc_out(sp, dinvp):
    def body(s_ref, dinv_ref, o_ref):
        o_ref[...] = jax.nn.sigmoid(dinv_ref[...] * (s_ref[0] + s_ref[1]))

    return pl.pallas_call(
        body,
        grid=(GRID_TC,),
        in_specs=[
            pl.BlockSpec((NC, ROWS_TC, 128), lambda i: (0, i, 0)),
            pl.BlockSpec((ROWS_TC, 128), lambda i: (i, 0)),
        ],
        out_specs=pl.BlockSpec((ROWS_TC, 128), lambda i: (i, 0)),
        out_shape=jax.ShapeDtypeStruct((NP8, 128), jnp.float32),
    )(sp, dinvp)


def kernel(x, edge_index, W1, b1, W2, b2):
    padm = (N + (jnp.arange(E_PAD - E, dtype=jnp.int32) % PAD_ROWS)).reshape(
        NROWS - E // CHUNK, CHUNK)
    dstm = jnp.concatenate([edge_index[1].reshape(E // CHUNK, CHUNK), padm])
    srcm = jnp.concatenate([edge_index[0].reshape(E // CHUNK, CHUNK), padm])
    x_p = jnp.pad(x, ((0, N_PAD - N), (0, 0)))
    w1w = jnp.tile(W1.T, (1, 8))
    col = lax.broadcasted_iota(jnp.int32, (8 * ROWS1, 128), 1)
    row = lax.broadcasted_iota(jnp.int32, (8 * ROWS1, 128), 0)
    mask_big = (col // 16 == row % 8).astype(jnp.float32)
    sr = lax.broadcasted_iota(jnp.int32, (ROWS1, 8 * ROWS1), 0)
    sc = lax.broadcasted_iota(jnp.int32, (ROWS1, 8 * ROWS1), 1)
    ssum = (sc // 8 == sr).astype(jnp.float32)
    bc = lax.broadcasted_iota(jnp.int32, (128, 128), 1)
    br = lax.broadcasted_iota(jnp.int32, (128, 128), 0)
    w2b = jnp.tile(W2.T, (8, 8)) * (bc // 16 == br // 16).astype(jnp.float32)
    b1b = jnp.tile(b1, 8)[None, :]
    b2b = jnp.tile(b2, 8)[None, :]

    degp = _sc_degree(dstm).reshape(NC, NP8, 128)
    g1p, dinvp = _tc_layer1(x_p, degp, w1w, mask_big, ssum, b1b)
    s1p = _sc_propagate(g1p.reshape(N_PAD, D_H), srcm, dstm).reshape(
        NC, NP8, 128)
    g2p = _tc_layer2(s1p, dinvp, w2b, b2b)
    s2p = _sc_propagate(g2p.reshape(N_PAD, D_H), srcm, dstm).reshape(
        NC, NP8, 128)
    outp = _tc_out(s2p, dinvp)
    return outp[:NOUT8].reshape(N, D_H)


# speedup vs baseline: 1.2949x; 1.0908x over previous
import functools

import jax
import jax.numpy as jnp
from jax import lax
from jax.experimental import pallas as pl
from jax.experimental.pallas import tpu as pltpu
from jax.experimental.pallas import tpu_sc as plsc

N = 10000
E = 320000
D_IN = 128
D_H = 16

NC = 2
NS = 16
NW = NC * NS
CHUNK = 128
CPT = 80
E_PAD = NW * CPT * CHUNK
NROWS = E_PAD // CHUNK
N_PAD = 10240
SLICE = N_PAD // NS
PAD_ROWS = 240
GRP = 8
NGRP = CPT // GRP
NP8 = N_PAD // 8
PSL = NP8 // NS
ROWS1 = 128
GRID1 = NP8 // ROWS1
ROWS_TC = 256
GRID_TC = NP8 // ROWS_TC
NOUT8 = N // 8

_mesh = plsc.VectorSubcoreMesh(core_axis_name="c", subcore_axis_name="s")

EROWS = E // CHUNK
TAIL = NW * CPT - EROWS
MAIN_LAST = CPT - TAIL


def _load_idx(ei3, padm, which, tid, idx_v):
    row0 = tid * CPT

    @pl.when(tid < NW - 1)
    def _():
        pltpu.sync_copy(ei3.at[which, pl.ds(row0, CPT)], idx_v)

    @pl.when(tid == NW - 1)
    def _():
        pltpu.sync_copy(ei3.at[which, pl.ds((NW - 1) * CPT, MAIN_LAST)],
                        idx_v.at[pl.ds(0, MAIN_LAST)])
        pltpu.sync_copy(padm, idx_v.at[pl.ds(MAIN_LAST, TAIL)])


@functools.partial(
    pl.kernel,
    out_type=jax.ShapeDtypeStruct((NC * NP8 * 128,), jnp.float32),
    mesh=_mesh,
    compiler_params=pltpu.CompilerParams(use_tc_tiling_on_sc=False),
    scratch_types=[
        pltpu.VMEM((CPT, CHUNK), jnp.int32),
        pltpu.VMEM((CHUNK,), jnp.float32),
        pltpu.VMEM((SLICE,), jnp.float32),
        pltpu.VMEM((SLICE * 16,), jnp.float32),
        pltpu.VMEM_SHARED((N_PAD,), jnp.float32),
        pltpu.SemaphoreType.DMA,
    ],
)
def _sc_degree(ei3, padm, out_hbm, dst_v, ones_v, stage, packed, acc, ssem):
    c = lax.axis_index("c")
    s = lax.axis_index("s")
    tid = c * NS + s
    _load_idx(ei3, padm, 1, tid, dst_v)

    def _fill(i, _):
        ones_v[pl.ds(i * 16, 16)] = jnp.ones((16,), jnp.float32)
        return 0

    lax.fori_loop(0, CHUNK // 16, _fill, 0)

    def _zero(i, _):
        stage[pl.ds(i * 16, 16)] = jnp.zeros((16,), jnp.float32)
        return 0

    lax.fori_loop(0, SLICE // 16, _zero, 0)
    pltpu.sync_copy(stage, acc.at[pl.ds(s * SLICE, SLICE)])
    plsc.subcore_barrier()

    def _group(gi, _):
        for b in range(8):
            pltpu.async_copy(ones_v, acc.at[dst_v.at[gi * 8 + b]], ssem,
                             add=True)
        for b in range(8):
            pltpu.make_async_copy(ones_v, acc.at[dst_v.at[gi * 8 + b]],
                                  ssem).wait()
        return 0

    lax.fori_loop(0, CPT // 8, _group, 0)
    plsc.subcore_barrier()
    pltpu.sync_copy(acc.at[pl.ds(s * SLICE, SLICE)], stage)

    def _expand(k, _):
        v = stage[pl.ds(k * 16, 16)]
        for l in range(16):
            bcast = v.at[jnp.full((16,), l, jnp.int32)].get(
                mode="promise_in_bounds")
            packed[pl.ds((k * 16 + l) * 16, 16)] = bcast
        return 0

    lax.fori_loop(0, SLICE // 16, _expand, 0)
    pltpu.sync_copy(packed,
                    out_hbm.at[pl.ds((c * N_PAD + s * SLICE) * 16, SLICE * 16)])


@functools.partial(
    pl.kernel,
    out_type=jax.ShapeDtypeStruct((NC * N_PAD, D_H), jnp.float32),
    mesh=_mesh,
    compiler_params=pltpu.CompilerParams(use_tc_tiling_on_sc=False),
    scratch_types=[
        pltpu.VMEM((CPT, CHUNK), jnp.int32),
        pltpu.VMEM((CPT, CHUNK), jnp.int32),
        [pltpu.VMEM((CHUNK, D_H), jnp.float32) for _ in range(2 * GRP)],
        pltpu.VMEM((SLICE, D_H), jnp.float32),
        pltpu.VMEM_SHARED((N_PAD, D_H), jnp.float32),
        pltpu.VMEM_SHARED((N_PAD, D_H), jnp.float32),
        [pltpu.SemaphoreType.DMA for _ in range(4)],
    ],
)
def _sc_propagate(g_hbm, ei3, padm, out_hbm, src_v, dst_v, bufs, stage, acc,
                  g_sh, sems):
    c = lax.axis_index("c")
    s = lax.axis_index("s")
    tid = c * NS + s
    _load_idx(ei3, padm, 0, tid, src_v)
    _load_idx(ei3, padm, 1, tid, dst_v)
    pltpu.sync_copy(g_hbm.at[pl.ds(s * SLICE, SLICE)],
                    g_sh.at[pl.ds(s * SLICE, SLICE)])

    def _zero(i, _):
        stage[i, :] = jnp.zeros((16,), jnp.float32)
        return 0

    lax.fori_loop(0, SLICE, _zero, 0)
    pltpu.sync_copy(stage, acc.at[pl.ds(s * SLICE, SLICE)])
    plsc.subcore_barrier()

    gsem = [sems[0], sems[1]]
    ssem = [sems[2], sems[3]]

    def _gather(j, buf, sem):
        pltpu.async_copy(g_sh.at[src_v.at[j]], buf, sem)

    def _gwait(j, buf, sem):
        pltpu.make_async_copy(g_sh.at[src_v.at[j]], buf, sem).wait()

    for b in range(GRP):
        _gather(b, bufs[b], gsem[0])
        _gather(GRP + b, bufs[GRP + b], gsem[1])

    def _group(m, _):
        for p in range(2):
            g0 = 2 * m + p
            sbufs = bufs[p * GRP:(p + 1) * GRP]
            for b in range(GRP):
                _gwait(g0 * GRP + b, sbufs[b], gsem[p])
            for b in range(GRP):
                pltpu.async_copy(sbufs[b], acc.at[dst_v.at[g0 * GRP + b]],
                                 ssem[p], add=True)
            for b in range(GRP):
                pltpu.make_async_copy(sbufs[b],
                                      acc.at[dst_v.at[g0 * GRP + b]],
                                      ssem[p]).wait()

            @pl.when(g0 + 2 < NGRP)
            def _():
                for b in range(GRP):
                    _gather((g0 + 2) * GRP + b, sbufs[b], gsem[p])

        return 0

    lax.fori_loop(0, NGRP // 2, _group, 0)
    plsc.subcore_barrier()
    pltpu.sync_copy(acc.at[pl.ds(s * SLICE, SLICE)], stage)
    pltpu.sync_copy(stage, out_hbm.at[pl.ds(c * N_PAD + s * SLICE, SLICE)])


def _tc_layer1(x_p, degp, w1w, mask_big, ssum, b1b):
    def body(x_ref, d_ref, w_ref, m_ref, s_ref, b_ref, g_ref, dinv_ref):
        deg = d_ref[0] + d_ref[1]
        dinv = jnp.where(deg > 0.0, lax.rsqrt(deg), 0.0)
        hw = jnp.dot(x_ref[...], w_ref[...], preferred_element_type=jnp.float32)
        hp = jnp.dot(s_ref[...], hw * m_ref[...],
                     preferred_element_type=jnp.float32)
        g_ref[...] = dinv * (hp + b_ref[...])
        dinv_ref[...] = dinv

    return pl.pallas_call(
        body,
        grid=(GRID1,),
        in_specs=[
            pl.BlockSpec((8 * ROWS1, D_IN), lambda i: (i, 0)),
            pl.BlockSpec((NC, ROWS1, 128), lambda i: (0, i, 0)),
            pl.BlockSpec((D_IN, 128), lambda i: (0, 0)),
            pl.BlockSpec((8 * ROWS1, 128), lambda i: (0, 0)),
            pl.BlockSpec((ROWS1, 8 * ROWS1), lambda i: (0, 0)),
            pl.BlockSpec((1, 128), lambda i: (0, 0)),
        ],
        out_specs=[
            pl.BlockSpec((ROWS1, 128), lambda i: (i, 0)),
            pl.BlockSpec((ROWS1, 128), lambda i: (i, 0)),
        ],
        out_shape=[
            jax.ShapeDtypeStruct((NP8, 128), jnp.float32),
            jax.ShapeDtypeStruct((NP8, 128), jnp.float32),
        ],
    )(x_p, degp, w1w, mask_big, ssum, b1b)


def _tc_layer2(sp, dinvp, w2b, b2b):
    def body(s_ref, dinv_ref, w_ref, b_ref, g_ref):
        a = jax.nn.sigmoid(dinv_ref[...] * (s_ref[0] + s_ref[1]))
        h = jnp.dot(a, w_ref[...], preferred_element_type=jnp.float32)
        g_ref[...] = dinv_ref[...] * (h + b_ref[...])

    return pl.pallas_call(
        body,
        grid=(GRID_TC,),
        in_specs=[
            pl.BlockSpec((NC, ROWS_TC, 128), lambda i: (0, i, 0)),
            pl.BlockSpec((ROWS_TC, 128), lambda i: (i, 0)),
            pl.BlockSpec((128, 128), lambda i: (0, 0)),
            pl.BlockSpec((1, 128), lambda i: (0, 0)),
        ],
        out_specs=pl.BlockSpec((ROWS_TC, 128), lambda i: (i, 0)),
        out_shape=jax.ShapeDtypeStruct((NP8, 128), jnp.float32),
    )(sp, dinvp, w2b, b2b)


def _tc_out(sp, dinvp):
    def body(s_ref, dinv_ref, o_ref):
        o_ref[...] = jax.nn.sigmoid(dinv_ref[...] * (s_ref[0] + s_ref[1]))

    return pl.pallas_call(
        body,
        grid=(GRID_TC,),
        in_specs=[
            pl.BlockSpec((NC, ROWS_TC, 128), lambda i: (0, i, 0)),
            pl.BlockSpec((ROWS_TC, 128), lambda i: (i, 0)),
        ],
        out_specs=pl.BlockSpec((ROWS_TC, 128), lambda i: (i, 0)),
        out_shape=jax.ShapeDtypeStruct((NP8, 128), jnp.float32),
    )(sp, dinvp)


def kernel(x, edge_index, W1, b1, W2, b2):
    ei3 = edge_index.reshape(2, EROWS, CHUNK)
    padm = (N + (jnp.arange(E_PAD - E, dtype=jnp.int32) % PAD_ROWS)).reshape(
        TAIL, CHUNK)
    x_p = jnp.pad(x, ((0, N_PAD - N), (0, 0)))
    w1w = jnp.tile(W1.T, (1, 8))
    col = lax.broadcasted_iota(jnp.int32, (8 * ROWS1, 128), 1)
    row = lax.broadcasted_iota(jnp.int32, (8 * ROWS1, 128), 0)
    mask_big = (col // 16 == row % 8).astype(jnp.float32)
    sr = lax.broadcasted_iota(jnp.int32, (ROWS1, 8 * ROWS1), 0)
    sc = lax.broadcasted_iota(jnp.int32, (ROWS1, 8 * ROWS1), 1)
    ssum = (sc // 8 == sr).astype(jnp.float32)
    bc = lax.broadcasted_iota(jnp.int32, (128, 128), 1)
    br = lax.broadcasted_iota(jnp.int32, (128, 128), 0)
    w2b = jnp.tile(W2.T, (8, 8)) * (bc // 16 == br // 16).astype(jnp.float32)
    b1b = jnp.tile(b1, 8)[None, :]
    b2b = jnp.tile(b2, 8)[None, :]

    degp = _sc_degree(ei3, padm).reshape(NC, NP8, 128)
    g1p, dinvp = _tc_layer1(x_p, degp, w1w, mask_big, ssum, b1b)
    s1p = _sc_propagate(g1p.reshape(N_PAD, D_H), ei3, padm).reshape(
        NC, NP8, 128)
    g2p = _tc_layer2(s1p, dinvp, w2b, b2b)
    s2p = _sc_propagate(g2p.reshape(N_PAD, D_H), ei3, padm).reshape(
        NC, NP8, 128)
    outp = _tc_out(s2p, dinvp)
    return outp[:NOUT8].reshape(N, D_H)


# speedup vs baseline: 1.3071x; 1.0095x over previous
import functools

import jax
import jax.numpy as jnp
from jax import lax
from jax.experimental import pallas as pl
from jax.experimental.pallas import tpu as pltpu
from jax.experimental.pallas import tpu_sc as plsc

N = 10000
E = 320000
D_IN = 128
D_H = 16

NC = 2
NS = 16
NW = NC * NS
CHUNK = 128
CPT = 80
E_PAD = NW * CPT * CHUNK
NROWS = E_PAD // CHUNK
N_PAD = 10240
SLICE = N_PAD // NS
PAD_ROWS = 240
GRP = 8
NGRP = CPT // GRP
NP8 = N_PAD // 8
PSL = NP8 // NS
ROWS1 = 256
GRID1 = NP8 // ROWS1
ROWS_TC = 256
GRID_TC = NP8 // ROWS_TC
NOUT8 = N // 8

_mesh = plsc.VectorSubcoreMesh(core_axis_name="c", subcore_axis_name="s")

EROWS = E // CHUNK
TAIL = NW * CPT - EROWS
MAIN_LAST = CPT - TAIL


def _load_idx(ei3, padm, which, tid, idx_v):
    row0 = tid * CPT

    @pl.when(tid < NW - 1)
    def _():
        pltpu.sync_copy(ei3.at[which, pl.ds(row0, CPT)], idx_v)

    @pl.when(tid == NW - 1)
    def _():
        pltpu.sync_copy(ei3.at[which, pl.ds((NW - 1) * CPT, MAIN_LAST)],
                        idx_v.at[pl.ds(0, MAIN_LAST)])
        pltpu.sync_copy(padm, idx_v.at[pl.ds(MAIN_LAST, TAIL)])


@functools.partial(
    pl.kernel,
    out_type=jax.ShapeDtypeStruct((NC * NP8 * 128,), jnp.float32),
    mesh=_mesh,
    compiler_params=pltpu.CompilerParams(use_tc_tiling_on_sc=False),
    scratch_types=[
        pltpu.VMEM((CPT, CHUNK), jnp.int32),
        pltpu.VMEM((CHUNK,), jnp.float32),
        pltpu.VMEM((SLICE,), jnp.float32),
        pltpu.VMEM((SLICE * 16,), jnp.float32),
        pltpu.VMEM_SHARED((N_PAD,), jnp.float32),
        pltpu.SemaphoreType.DMA,
    ],
)
def _sc_degree(ei3, padm, out_hbm, dst_v, ones_v, stage, packed, acc, ssem):
    c = lax.axis_index("c")
    s = lax.axis_index("s")
    tid = c * NS + s
    _load_idx(ei3, padm, 1, tid, dst_v)

    def _fill(i, _):
        ones_v[pl.ds(i * 16, 16)] = jnp.ones((16,), jnp.float32)
        return 0

    lax.fori_loop(0, CHUNK // 16, _fill, 0)

    def _zero(i, _):
        stage[pl.ds(i * 16, 16)] = jnp.zeros((16,), jnp.float32)
        return 0

    lax.fori_loop(0, SLICE // 16, _zero, 0)
    pltpu.sync_copy(stage, acc.at[pl.ds(s * SLICE, SLICE)])
    plsc.subcore_barrier()

    def _group(gi, _):
        for b in range(8):
            pltpu.async_copy(ones_v, acc.at[dst_v.at[gi * 8 + b]], ssem,
                             add=True)
        for b in range(8):
            pltpu.make_async_copy(ones_v, acc.at[dst_v.at[gi * 8 + b]],
                                  ssem).wait()
        return 0

    lax.fori_loop(0, CPT // 8, _group, 0)
    plsc.subcore_barrier()
    pltpu.sync_copy(acc.at[pl.ds(s * SLICE, SLICE)], stage)

    def _expand(k, _):
        v = stage[pl.ds(k * 16, 16)]
        for l in range(16):
            bcast = v.at[jnp.full((16,), l, jnp.int32)].get(
                mode="promise_in_bounds")
            packed[pl.ds((k * 16 + l) * 16, 16)] = bcast
        return 0

    lax.fori_loop(0, SLICE // 16, _expand, 0)
    pltpu.sync_copy(packed,
                    out_hbm.at[pl.ds((c * N_PAD + s * SLICE) * 16, SLICE * 16)])


@functools.partial(
    pl.kernel,
    out_type=jax.ShapeDtypeStruct((NC * N_PAD, D_H), jnp.float32),
    mesh=_mesh,
    compiler_params=pltpu.CompilerParams(use_tc_tiling_on_sc=False),
    scratch_types=[
        pltpu.VMEM((CPT, CHUNK), jnp.int32),
        pltpu.VMEM((CPT, CHUNK), jnp.int32),
        [pltpu.VMEM((CHUNK, D_H), jnp.float32) for _ in range(2 * GRP)],
        pltpu.VMEM((SLICE, D_H), jnp.float32),
        pltpu.VMEM_SHARED((N_PAD, D_H), jnp.float32),
        pltpu.VMEM_SHARED((N_PAD, D_H), jnp.float32),
        [pltpu.SemaphoreType.DMA for _ in range(4)],
    ],
)
def _sc_propagate(g_hbm, ei3, padm, out_hbm, src_v, dst_v, bufs, stage, acc,
                  g_sh, sems):
    c = lax.axis_index("c")
    s = lax.axis_index("s")
    tid = c * NS + s
    _load_idx(ei3, padm, 0, tid, src_v)
    _load_idx(ei3, padm, 1, tid, dst_v)
    pltpu.sync_copy(g_hbm.at[pl.ds(s * SLICE, SLICE)],
                    g_sh.at[pl.ds(s * SLICE, SLICE)])

    def _zero(i, _):
        stage[i, :] = jnp.zeros((16,), jnp.float32)
        return 0

    lax.fori_loop(0, SLICE, _zero, 0)
    pltpu.sync_copy(stage, acc.at[pl.ds(s * SLICE, SLICE)])
    plsc.subcore_barrier()

    gsem = [sems[0], sems[1]]
    ssem = [sems[2], sems[3]]

    def _gather(j, buf, sem):
        pltpu.async_copy(g_sh.at[src_v.at[j]], buf, sem)

    def _gwait(j, buf, sem):
        pltpu.make_async_copy(g_sh.at[src_v.at[j]], buf, sem).wait()

    for b in range(GRP):
        _gather(b, bufs[b], gsem[0])
        _gather(GRP + b, bufs[GRP + b], gsem[1])

    def _group(m, _):
        for p in range(2):
            g0 = 2 * m + p
            sbufs = bufs[p * GRP:(p + 1) * GRP]
            for b in range(GRP):
                _gwait(g0 * GRP + b, sbufs[b], gsem[p])
            for b in range(GRP):
                pltpu.async_copy(sbufs[b], acc.at[dst_v.at[g0 * GRP + b]],
                                 ssem[p], add=True)
            for b in range(GRP):
                pltpu.make_async_copy(sbufs[b],
                                      acc.at[dst_v.at[g0 * GRP + b]],
                                      ssem[p]).wait()

            @pl.when(g0 + 2 < NGRP)
            def _():
                for b in range(GRP):
                    _gather((g0 + 2) * GRP + b, sbufs[b], gsem[p])

        return 0

    lax.fori_loop(0, NGRP // 2, _group, 0)
    plsc.subcore_barrier()
    pltpu.sync_copy(acc.at[pl.ds(s * SLICE, SLICE)], stage)
    pltpu.sync_copy(stage, out_hbm.at[pl.ds(c * N_PAD + s * SLICE, SLICE)])


def _tc_layer1(x_p, degp, w1w, mask_big, ssum, b1b):
    def body(x_ref, d_ref, w_ref, m_ref, s_ref, b_ref, g_ref, dinv_ref):
        deg = d_ref[0] + d_ref[1]
        dinv = jnp.where(deg > 0.0, lax.rsqrt(deg), 0.0)
        hw = jnp.dot(x_ref[...], w_ref[...], preferred_element_type=jnp.float32)
        hp = jnp.dot(s_ref[...], hw * m_ref[...],
                     preferred_element_type=jnp.float32)
        g_ref[...] = dinv * (hp + b_ref[...])
        dinv_ref[...] = dinv

    return pl.pallas_call(
        body,
        grid=(GRID1,),
        in_specs=[
            pl.BlockSpec((8 * ROWS1, D_IN), lambda i: (i, 0)),
            pl.BlockSpec((NC, ROWS1, 128), lambda i: (0, i, 0)),
            pl.BlockSpec((D_IN, 128), lambda i: (0, 0)),
            pl.BlockSpec((8 * ROWS1, 128), lambda i: (0, 0)),
            pl.BlockSpec((ROWS1, 8 * ROWS1), lambda i: (0, 0)),
            pl.BlockSpec((1, 128), lambda i: (0, 0)),
        ],
        out_specs=[
            pl.BlockSpec((ROWS1, 128), lambda i: (i, 0)),
            pl.BlockSpec((ROWS1, 128), lambda i: (i, 0)),
        ],
        out_shape=[
            jax.ShapeDtypeStruct((NP8, 128), jnp.float32),
            jax.ShapeDtypeStruct((NP8, 128), jnp.float32),
        ],
    )(x_p, degp, w1w, mask_big, ssum, b1b)


def _tc_layer2(sp, dinvp, w2b, b2b):
    def body(s_ref, dinv_ref, w_ref, b_ref, g_ref):
        a = jax.nn.sigmoid(dinv_ref[...] * (s_ref[0] + s_ref[1]))
        h = jnp.dot(a, w_ref[...], preferred_element_type=jnp.float32)
        g_ref[...] = dinv_ref[...] * (h + b_ref[...])

    return pl.pallas_call(
        body,
        grid=(GRID_TC,),
        in_specs=[
            pl.BlockSpec((NC, ROWS_TC, 128), lambda i: (0, i, 0)),
            pl.BlockSpec((ROWS_TC, 128), lambda i: (i, 0)),
            pl.BlockSpec((128, 128), lambda i: (0, 0)),
            pl.BlockSpec((1, 128), lambda i: (0, 0)),
        ],
        out_specs=pl.BlockSpec((ROWS_TC, 128), lambda i: (i, 0)),
        out_shape=jax.ShapeDtypeStruct((NP8, 128), jnp.float32),
    )(sp, dinvp, w2b, b2b)


def _tc_out(sp, dinvp):
    def body(s_ref, dinv_ref, o_ref):
        o_ref[...] = jax.nn.sigmoid(dinv_ref[...] * (s_ref[0] + s_ref[1]))

    return pl.pallas_call(
        body,
        grid=(GRID_TC,),
        in_specs=[
            pl.BlockSpec((NC, ROWS_TC, 128), lambda i: (0, i, 0)),
            pl.BlockSpec((ROWS_TC, 128), lambda i: (i, 0)),
        ],
        out_specs=pl.BlockSpec((ROWS_TC, 128), lambda i: (i, 0)),
        out_shape=jax.ShapeDtypeStruct((NP8, 128), jnp.float32),
    )(sp, dinvp)


def kernel(x, edge_index, W1, b1, W2, b2):
    ei3 = edge_index.reshape(2, EROWS, CHUNK)
    padm = (N + (jnp.arange(E_PAD - E, dtype=jnp.int32) % PAD_ROWS)).reshape(
        TAIL, CHUNK)
    x_p = jnp.pad(x, ((0, N_PAD - N), (0, 0)))
    w1w = jnp.tile(W1.T, (1, 8))
    col = lax.broadcasted_iota(jnp.int32, (8 * ROWS1, 128), 1)
    row = lax.broadcasted_iota(jnp.int32, (8 * ROWS1, 128), 0)
    mask_big = (col // 16 == row % 8).astype(jnp.float32)
    sr = lax.broadcasted_iota(jnp.int32, (ROWS1, 8 * ROWS1), 0)
    sc = lax.broadcasted_iota(jnp.int32, (ROWS1, 8 * ROWS1), 1)
    ssum = (sc // 8 == sr).astype(jnp.float32)
    bc = lax.broadcasted_iota(jnp.int32, (128, 128), 1)
    br = lax.broadcasted_iota(jnp.int32, (128, 128), 0)
    w2b = jnp.tile(W2.T, (8, 8)) * (bc // 16 == br // 16).astype(jnp.float32)
    b1b = jnp.tile(b1, 8)[None, :]
    b2b = jnp.tile(b2, 8)[None, :]

    degp = _sc_degree(ei3, padm).reshape(NC, NP8, 128)
    g1p, dinvp = _tc_layer1(x_p, degp, w1w, mask_big, ssum, b1b)
    s1p = _sc_propagate(g1p.reshape(N_PAD, D_H), ei3, padm).reshape(
        NC, NP8, 128)
    g2p = _tc_layer2(s1p, dinvp, w2b, b2b)
    s2p = _sc_propagate(g2p.reshape(N_PAD, D_H), ei3, padm).reshape(
        NC, NP8, 128)
    outp = _tc_out(s2p, dinvp)
    return outp[:NOUT8].reshape(N, D_H)
